# baseline (speedup 1.0000x reference)
import functools

import jax
import jax.numpy as jnp
import numpy as np
from jax.experimental import pallas as pl
from jax.experimental.pallas import tpu as pltpu

_N_INPUT = 3
_N_LATENT = 8
_D = _N_INPUT + _N_LATENT
_D1 = _D + 1
_N_LEVELS = 16
_N_FEATS = 2
_T = 1 << 18
_COARSEST = 16.0
_FINEST = 2048.0
_N_OUT = 4
_WH = 64
_N_PTS = 131072

_PRIMES = [1, 2654435761, 805459861, 3674653429, 2097192037, 1434869437,
           2165219737, 2654435741, 2246822519, 3266489917, 668265263]


def _build_E(d):
    E = np.zeros((d + 1, d), dtype=np.float64)
    for k in range(d):
        cf = np.zeros(d)
        cf[k] = 1.0
        e = np.zeros(d + 1)
        e[d] = -d * cf[d - 1]
        for i in range(d - 1, 0, -1):
            e[i] = e[i + 1] - i * cf[i - 1] + (i + 2) * cf[i]
        e[0] = e[1] + 2.0 * cf[0]
        E[:, k] = e
    return E


_SF = 1.0 / np.sqrt((np.arange(_D) + 1.0) * (np.arange(_D) + 2.0))
_ET = _build_E(_D).T
_SCALES = _COARSEST * (_FINEST / _COARSEST) ** (np.arange(_N_LEVELS) / (_N_LEVELS - 1.0))


def _permuto_level(elev):
    greedy = jnp.round(elev / _D1) * _D1
    diff = elev - greedy
    A = diff[:, :, None] < diff[:, None, :]
    triu = jnp.triu(jnp.ones((_D1, _D1), bool), 1)
    tril = jnp.tril(jnp.ones((_D1, _D1), bool), -1)
    rank = (A & triu).sum(-1).astype(jnp.int32) + ((~jnp.swapaxes(A, -1, -2)) & tril).sum(-1).astype(jnp.int32)
    s = jnp.round(greedy.sum(-1) / _D1).astype(jnp.int32)
    rank = rank + s[:, None]
    lo = rank < 0
    hi = rank > _D
    rank = jnp.where(lo, rank + _D1, jnp.where(hi, rank - _D1, rank))
    greedy = jnp.where(lo, greedy + _D1, jnp.where(hi, greedy - _D1, greedy))
    t = (elev - greedy) / _D1
    bary = (jax.nn.one_hot(_D - rank, _D + 2, dtype=elev.dtype) * t[..., None]).sum(-2)
    bary = bary - (jax.nn.one_hot(_D + 1 - rank, _D + 2, dtype=elev.dtype) * t[..., None]).sum(-2)
    bary = bary.at[:, 0].add(1.0 + bary[:, _D + 1])
    return greedy, rank, bary


def _hash(keys_u32):
    h = jnp.zeros(keys_u32.shape[:-1], jnp.uint32)
    for i in range(_D):
        h = h ^ (keys_u32[..., i] * jnp.uint32(_PRIMES[i]))
    return (h % jnp.uint32(_T)).astype(jnp.int32)


def _encode(pos, tables):
    base = (pos * jnp.asarray(_SF, pos.dtype)) @ jnp.asarray(_ET, pos.dtype)
    outs = []
    for l in range(_N_LEVELS):
        elev = base * jnp.asarray(_SCALES[l], pos.dtype)
        greedy, rank, bary = _permuto_level(elev)
        acc = jnp.zeros((pos.shape[0], _N_FEATS), pos.dtype)
        for r in range(_D1):
            key = (greedy + r - _D1 * (rank > (_D - r)).astype(pos.dtype)).astype(jnp.int32)
            idx = _hash(key.astype(jnp.uint32))
            acc = acc + bary[:, r:r + 1] * tables[l][idx]
        outs.append(acc)
    return jnp.concatenate(outs, axis=-1)


def _mlp_body(h_ref, w0_ref, b0_ref, w1_ref, b1_ref, w2_ref, b2_ref, o_ref):
    h = h_ref[...]
    a = jnp.maximum(jnp.dot(h, w0_ref[...], preferred_element_type=jnp.float32) + b0_ref[...], 0.0)
    a = jnp.maximum(jnp.dot(a, w1_ref[...], preferred_element_type=jnp.float32) + b1_ref[...], 0.0)
    o_ref[...] = jnp.dot(a, w2_ref[...], preferred_element_type=jnp.float32) + b2_ref[...]


@functools.partial(jax.jit, static_argnames=())
def _mlp(h, W0, b0, W1, b1, W2, b2):
    n = h.shape[0]
    blk = 2048
    grid = (n // blk,)
    return pl.pallas_call(
        _mlp_body,
        grid=grid,
        in_specs=[
            pl.BlockSpec((blk, 2 * _N_LEVELS), lambda i: (i, 0)),
            pl.BlockSpec((2 * _N_LEVELS, _WH), lambda i: (0, 0)),
            pl.BlockSpec((1, _WH), lambda i: (0, 0)),
            pl.BlockSpec((_WH, _WH), lambda i: (0, 0)),
            pl.BlockSpec((1, _WH), lambda i: (0, 0)),
            pl.BlockSpec((_WH, _N_OUT), lambda i: (0, 0)),
            pl.BlockSpec((1, _N_OUT), lambda i: (0, 0)),
        ],
        out_specs=pl.BlockSpec((blk, _N_OUT), lambda i: (i, 0)),
        out_shape=jax.ShapeDtypeStruct((n, _N_OUT), jnp.float32),
    )(h, W0, b0.reshape(1, -1), W1, b1.reshape(1, -1), W2, b2.reshape(1, -1))


def kernel(x, bidx, z_per_batch, tables, W0, b0, W1, b1, W2, b2):
    z = z_per_batch[bidx]
    pos = jnp.concatenate([x, z], axis=-1)
    h = _encode(pos, tables)
    return _mlp(h, W0, b0, W1, b1, W2, b2)


# speedup vs baseline: 2.3324x; 2.3324x over previous
import jax
import jax.numpy as jnp
import numpy as np
from jax import lax
from jax.experimental import pallas as pl
from jax.experimental.pallas import tpu as pltpu
from jax.experimental.pallas import tpu_sc as plsc

_D = 11
_D1 = 12
_NL = 16
_T = 1 << 18
_NPTS = 131072
_WH = 64
_NOUT = 4

_PRIMES = [1, 2654435761, 805459861, 3674653429, 2097192037, 1434869437,
           2165219737, 2654435741, 2246822519, 3266489917, 668265263]


def _build_E(d):
    E = np.zeros((d + 1, d), dtype=np.float64)
    for k in range(d):
        cf = np.zeros(d)
        cf[k] = 1.0
        e = np.zeros(d + 1)
        e[d] = -d * cf[d - 1]
        for i in range(d - 1, 0, -1):
            e[i] = e[i + 1] - i * cf[i - 1] + (i + 2) * cf[i]
        e[0] = e[1] + 2.0 * cf[0]
        E[:, k] = e
    return E


def _bf16_np(x):
    u = np.asarray(x, np.float32).view(np.uint32)
    r = (u + 0x7FFF + ((u >> 16) & 1)) & 0xFFFF0000
    return r.astype(np.uint32).view(np.float32)


_SF = 1.0 / np.sqrt((np.arange(_D) + 1.0) * (np.arange(_D) + 2.0))
_ET = _build_E(_D).T
_ETB = _bf16_np(_ET.astype(np.float32))
_SCALES = 16.0 * (2048.0 / 16.0) ** (np.arange(_NL) / (_NL - 1.0))
_MAGIC = np.float32(12582912.0)

_NW = 32
_PPW = _NPTS // _NW
_CP = 256
_CH = _PPW // _CP
_NG = _CP // 16
_NTR = _CP * _D1 // 128


def _i32(v):
    v = int(v) % (1 << 32)
    return v - (1 << 32) if v >= (1 << 31) else v


def _bf16_round(x):
    u = lax.bitcast_convert_type(x, jnp.int32)
    r = (u + 0x7FFF + (lax.shift_right_logical(u, 16) & 1)) & _i32(0xFFFF0000)
    return lax.bitcast_convert_type(r, jnp.float32)


def _sc_body(xT, bidx, zT, tab16, out, xc, bc, zt, basebuf, idxbuf, lobuf,
             wbuf, rows, ubuf, hc, gsem):
    wid = lax.axis_index("s") * 2 + lax.axis_index("c")
    iota = lax.iota(jnp.int32, 16)
    izero = jnp.full((16,), 0, jnp.int32)
    ione = jnp.full((16,), 1, jnp.int32)
    i12 = jnp.full((16,), 12, jnp.int32)
    im12 = jnp.full((16,), -12, jnp.int32)

    pltpu.sync_copy(zT, zt)

    def chunk_body(ch, carry):
        base_pt = wid * _PPW + ch * _CP
        pltpu.sync_copy(xT.at[:, pl.ds(base_pt, _CP)], xc)
        pltpu.sync_copy(bidx.at[pl.ds(base_pt, _CP)], bc)

        def bb(g, c2):
            sl = g * 16
            bv = bc[pl.ds(sl, 16)]
            ps = []
            for k in range(3):
                ps.append(_bf16_round(xc[k, pl.ds(sl, 16)] * np.float32(_SF[k])))
            for k in range(8):
                zk = plsc.load_gather(zt, [jnp.full((16,), k, jnp.int32), bv])
                ps.append(_bf16_round(zk * np.float32(_SF[3 + k])))
            for dd in range(_D1):
                acc = ps[0] * np.float32(_ETB[0, dd])
                for k in range(1, _D):
                    acc = acc + ps[k] * np.float32(_ETB[k, dd])
                basebuf[dd, pl.ds(sl, 16)] = acc
            return c2

        lax.fori_loop(0, _NG, bb, 0)

        def level_body(lev, c3):
            sc = np.float32(_SCALES[0])
            for l in range(1, _NL):
                sc = jnp.where(lev == l, np.float32(_SCALES[l]), sc)
            lev_off = lev * _T

            def pa(g, c4):
                sl = g * 16
                b = [basebuf[dd, pl.ds(sl, 16)] for dd in range(_D1)]
                elev = [bd * sc for bd in b]
                q = [e / np.float32(12.0) for e in elev]
                rq = [(qq + _MAGIC) - _MAGIC for qq in q]
                greedy = [r_ * np.float32(12.0) for r_ in rq]
                diff = [elev[d] - greedy[d] for d in range(_D1)]
                rank = [jnp.full((16,), d, jnp.int32) for d in range(_D1)]
                for i in range(_D1):
                    for j in range(i + 1, _D1):
                        ci = jnp.where(diff[i] < diff[j], ione, izero)
                        rank[i] = rank[i] + ci
                        rank[j] = rank[j] - ci
                gsum = greedy[0]
                for d in range(1, _D1):
                    gsum = gsum + greedy[d]
                qs = gsum / np.float32(12.0)
                sv = ((qs + _MAGIC) - _MAGIC).astype(jnp.int32)
                t = [None] * _D1
                for d in range(_D1):
                    rk = rank[d] + sv
                    adj = jnp.where(rk < 0, i12, jnp.where(rk > _D, im12, izero))
                    rank[d] = rk + adj
                    greedy[d] = greedy[d] + adj.astype(jnp.float32)
                    t[d] = (elev[d] - greedy[d]) * np.float32(1.0 / 12.0)
                for d in range(_D1):
                    uidx = (rank[d] & 15) * 16 + iota
                    plsc.store_scatter(ubuf, [uidx], t[d])
                u = [ubuf[pl.ds(k * 16, 16)] for k in range(_D1)]
                w0 = (np.float32(1.0) + u[_D]) - u[0]
                wbuf[0, pl.ds(sl, 16)] = w0
                for r in range(1, _D1):
                    wbuf[r, pl.ds(sl, 16)] = u[_D - r] - u[_D1 - r]
                gi = [greedy[d].astype(jnp.int32) for d in range(_D)]
                P = [gi[0]] + [gi[d] * np.int32(_i32(_PRIMES[d]))
                               for d in range(1, _D)]
                for r in range(_D1):
                    hsh = None
                    for d in range(_D):
                        if r == 0:
                            term = P[d]
                        else:
                            a_c = _i32(r * _PRIMES[d])
                            b_c = _i32((r - 12) * _PRIMES[d])
                            sel = jnp.where(rank[d] > (_D - r),
                                            jnp.full((16,), b_c, jnp.int32),
                                            jnp.full((16,), a_c, jnp.int32))
                            term = P[d] + sel
                        hsh = term if hsh is None else hsh ^ term
                    idxv = (hsh & np.int32(_T - 1)) + lev_off
                    p = sl * 12 + r * 16
                    idxbuf[pl.ds(p, 16)] = lax.shift_right_logical(idxv, 3)
                    lobuf[pl.ds(p, 16)] = idxv & 7
                return c4

            lax.fori_loop(0, _NG, pa, 0)

            def gb(jb, c5):
                for u_ in range(2):
                    j = jb * 2 + u_
                    pltpu.async_copy(tab16.at[idxbuf.at[pl.ds(j * 128, 128)]],
                                     rows.at[j], gsem)
                for u_ in range(2):
                    j = jb * 2 + u_
                    pltpu.make_async_copy(tab16.at[idxbuf.at[pl.ds(j * 128, 128)]],
                                          rows.at[j], gsem).wait()
                return c5

            lax.fori_loop(0, _NTR // 2, gb, 0)

            def pc(g, c6):
                sl = g * 16
                acc0 = jnp.zeros((16,), jnp.float32)
                acc1 = jnp.zeros((16,), jnp.float32)
                for r in range(_D1):
                    wv = wbuf[r, pl.ds(sl, 16)]
                    p = sl * 12 + r * 16
                    lov2 = lobuf[pl.ds(p, 16)] * 2
                    i0 = jnp.full((16,), p // 128, jnp.int32)
                    i1 = iota + (p % 128)
                    f0 = plsc.load_gather(rows, [i0, i1, lov2])
                    f1 = plsc.load_gather(rows, [i0, i1, lov2 + 1])
                    acc0 = acc0 + wv * f0
                    acc1 = acc1 + wv * f1
                hc[0, pl.ds(sl, 16)] = acc0
                hc[1, pl.ds(sl, 16)] = acc1
                return c6

            lax.fori_loop(0, _NG, pc, 0)
            pltpu.sync_copy(hc, out.at[pl.ds(lev * 2, 2), pl.ds(base_pt, _CP)])
            return c3

        lax.fori_loop(0, _NL, level_body, 0)
        return carry

    lax.fori_loop(0, _CH, chunk_body, 0)


def _sc_encode(x, bidx, z_per_batch, tables):
    xT = x.T
    zT = z_per_batch.T
    tab16 = tables.reshape(_NL * _T // 8, 16)
    mesh = plsc.VectorSubcoreMesh(core_axis_name="c", subcore_axis_name="s")
    kern = pl.kernel(
        _sc_body,
        out_type=jax.ShapeDtypeStruct((2 * _NL, _NPTS), jnp.float32),
        mesh=mesh,
        scratch_types=[
            pltpu.VMEM((3, _CP), jnp.float32),
            pltpu.VMEM((_CP,), jnp.int32),
            pltpu.VMEM((8, 16), jnp.float32),
            pltpu.VMEM((_D1, _CP), jnp.float32),
            pltpu.VMEM((_NTR * 128,), jnp.int32),
            pltpu.VMEM((_NTR * 128,), jnp.int32),
            pltpu.VMEM((_D1, _CP), jnp.float32),
            pltpu.VMEM((_NTR, 128, 16), jnp.float32),
            pltpu.VMEM((256,), jnp.float32),
            pltpu.VMEM((2, _CP), jnp.float32),
            pltpu.SemaphoreType.DMA,
        ],
        compiler_params=pltpu.CompilerParams(needs_layout_passes=False,
                                             use_tc_tiling_on_sc=False),
    )
    return kern(xT, bidx, zT, tab16)


def _mlp_body(h_ref, w0_ref, b0_ref, w1_ref, b1_ref, w2_ref, b2_ref, o_ref):
    hb = h_ref[...]
    a = lax.dot_general(hb, w0_ref[...], (((0,), (0,)), ((), ())),
                        preferred_element_type=jnp.float32) + b0_ref[...]
    a = jnp.maximum(a, 0.0)
    a = jnp.dot(a, w1_ref[...], preferred_element_type=jnp.float32) + b1_ref[...]
    a = jnp.maximum(a, 0.0)
    o_ref[...] = jnp.dot(a, w2_ref[...],
                         preferred_element_type=jnp.float32) + b2_ref[...]


def _mlp_t(hT, W0, b0, W1, b1, W2, b2):
    n = hT.shape[1]
    blk = 2048
    return pl.pallas_call(
        _mlp_body,
        grid=(n // blk,),
        in_specs=[
            pl.BlockSpec((2 * _NL, blk), lambda i: (0, i)),
            pl.BlockSpec((2 * _NL, _WH), lambda i: (0, 0)),
            pl.BlockSpec((1, _WH), lambda i: (0, 0)),
            pl.BlockSpec((_WH, _WH), lambda i: (0, 0)),
            pl.BlockSpec((1, _WH), lambda i: (0, 0)),
            pl.BlockSpec((_WH, _NOUT), lambda i: (0, 0)),
            pl.BlockSpec((1, _NOUT), lambda i: (0, 0)),
        ],
        out_specs=pl.BlockSpec((blk, _NOUT), lambda i: (i, 0)),
        out_shape=jax.ShapeDtypeStruct((n, _NOUT), jnp.float32),
    )(hT, W0, b0.reshape(1, -1), W1, b1.reshape(1, -1), W2, b2.reshape(1, -1))


def kernel(x, bidx, z_per_batch, tables, W0, b0, W1, b1, W2, b2):
    hT = _sc_encode(x, bidx, z_per_batch, tables)
    return _mlp_t(hT, W0, b0, W1, b1, W2, b2)


# speedup vs baseline: 3.2376x; 1.3881x over previous
import jax
import jax.numpy as jnp
import numpy as np
from jax import lax
from jax.experimental import pallas as pl
from jax.experimental.pallas import tpu as pltpu
from jax.experimental.pallas import tpu_sc as plsc

_D = 11
_D1 = 12
_NL = 16
_T = 1 << 18
_NPTS = 131072
_WH = 64
_NOUT = 4

_PRIMES = [1, 2654435761, 805459861, 3674653429, 2097192037, 1434869437,
           2165219737, 2654435741, 2246822519, 3266489917, 668265263]


def _build_E(d):
    E = np.zeros((d + 1, d), dtype=np.float64)
    for k in range(d):
        cf = np.zeros(d)
        cf[k] = 1.0
        e = np.zeros(d + 1)
        e[d] = -d * cf[d - 1]
        for i in range(d - 1, 0, -1):
            e[i] = e[i + 1] - i * cf[i - 1] + (i + 2) * cf[i]
        e[0] = e[1] + 2.0 * cf[0]
        E[:, k] = e
    return E


def _bf16_np(x):
    u = np.asarray(x, np.float32).view(np.uint32)
    r = (u + 0x7FFF + ((u >> 16) & 1)) & 0xFFFF0000
    return r.astype(np.uint32).view(np.float32)


_SF = 1.0 / np.sqrt((np.arange(_D) + 1.0) * (np.arange(_D) + 2.0))
_ET = _build_E(_D).T
_ETB = _bf16_np(_ET.astype(np.float32))
_SCALES = 16.0 * (2048.0 / 16.0) ** (np.arange(_NL) / (_NL - 1.0))
_MAGIC = np.float32(12582912.0)

_NW = 32
_PPW = _NPTS // _NW
_CP = 256
_CH = _PPW // _CP
_NG = _CP // 16
_NTR = _CP * _D1 // 128


def _i32(v):
    v = int(v) % (1 << 32)
    return v - (1 << 32) if v >= (1 << 31) else v


def _bf16_round(x):
    u = lax.bitcast_convert_type(x, jnp.int32)
    r = (u + 0x7FFF + (lax.shift_right_logical(u, 16) & 1)) & _i32(0xFFFF0000)
    return lax.bitcast_convert_type(r, jnp.float32)


def _sc_body(xT, bidx, zT, tab16, out, xc, bc, zt, basebuf, idxbuf, lobuf,
             wbuf, rows, ubuf, hc, gsem):
    wid = lax.axis_index("s") * 2 + lax.axis_index("c")
    iota = lax.iota(jnp.int32, 16)
    izero = jnp.full((16,), 0, jnp.int32)
    ione = jnp.full((16,), 1, jnp.int32)
    i12 = jnp.full((16,), 12, jnp.int32)
    im12 = jnp.full((16,), -12, jnp.int32)

    pltpu.sync_copy(zT, zt)

    def chunk_body(ch, carry):
        base_pt = wid * _PPW + ch * _CP
        pltpu.sync_copy(xT.at[:, pl.ds(base_pt, _CP)], xc)
        pltpu.sync_copy(bidx.at[pl.ds(base_pt, _CP)], bc)

        def bb(g, c2):
            sl = g * 16
            bv = bc[pl.ds(sl, 16)]
            ps = []
            for k in range(3):
                ps.append(_bf16_round(xc[k, pl.ds(sl, 16)] * np.float32(_SF[k])))
            for k in range(8):
                zk = plsc.load_gather(zt, [jnp.full((16,), k, jnp.int32), bv])
                ps.append(_bf16_round(zk * np.float32(_SF[3 + k])))
            for dd in range(_D1):
                acc = ps[0] * np.float32(_ETB[0, dd])
                for k in range(1, _D):
                    acc = acc + ps[k] * np.float32(_ETB[k, dd])
                basebuf[dd, pl.ds(sl, 16)] = acc
            return c2

        lax.fori_loop(0, _NG, bb, 0)

        def level_body(lev, c3):
            sc = np.float32(_SCALES[0])
            for l in range(1, _NL):
                sc = jnp.where(lev == l, np.float32(_SCALES[l]), sc)
            lev_off = lev * _T

            def pa_one(g):
                sl = g * 16
                b = [basebuf[dd, pl.ds(sl, 16)] for dd in range(_D1)]
                elev = [bd * sc for bd in b]
                q = [e / np.float32(12.0) for e in elev]
                rq = [(qq + _MAGIC) - _MAGIC for qq in q]
                greedy = [r_ * np.float32(12.0) for r_ in rq]
                diff = [elev[d] - greedy[d] for d in range(_D1)]
                rank = [jnp.full((16,), d, jnp.int32) for d in range(_D1)]
                for i in range(_D1):
                    for j in range(i + 1, _D1):
                        ci = jnp.where(diff[i] < diff[j], ione, izero)
                        rank[i] = rank[i] + ci
                        rank[j] = rank[j] - ci
                gsum = greedy[0]
                for d in range(1, _D1):
                    gsum = gsum + greedy[d]
                qs = gsum / np.float32(12.0)
                sv = ((qs + _MAGIC) - _MAGIC).astype(jnp.int32)
                t = [None] * _D1
                for d in range(_D1):
                    rk = rank[d] + sv
                    adj = jnp.where(rk < 0, i12, jnp.where(rk > _D, im12, izero))
                    rank[d] = rk + adj
                    greedy[d] = greedy[d] + adj.astype(jnp.float32)
                    t[d] = (elev[d] - greedy[d]) * np.float32(1.0 / 12.0)
                for d in range(_D1):
                    uidx = (rank[d] & 15) * 16 + iota
                    plsc.store_scatter(ubuf, [uidx], t[d])
                u = [ubuf[pl.ds(k * 16, 16)] for k in range(_D1)]
                w0 = (np.float32(1.0) + u[_D]) - u[0]
                wbuf[0, pl.ds(sl, 16)] = w0
                for r in range(1, _D1):
                    wbuf[r, pl.ds(sl, 16)] = u[_D - r] - u[_D1 - r]
                gi = [greedy[d].astype(jnp.int32) for d in range(_D)]
                P = [gi[0]] + [gi[d] * np.int32(_i32(_PRIMES[d]))
                               for d in range(1, _D)]
                for r in range(_D1):
                    hsh = None
                    for d in range(_D):
                        if r == 0:
                            term = P[d]
                        else:
                            a_c = _i32(r * _PRIMES[d])
                            b_c = _i32((r - 12) * _PRIMES[d])
                            sel = jnp.where(rank[d] > (_D - r),
                                            jnp.full((16,), b_c, jnp.int32),
                                            jnp.full((16,), a_c, jnp.int32))
                            term = P[d] + sel
                        hsh = term if hsh is None else hsh ^ term
                    idxv = (hsh & np.int32(_T - 1)) + lev_off
                    p = sl * 12 + r * 16
                    idxbuf[pl.ds(p, 16)] = lax.shift_right_logical(idxv, 3)
                    lobuf[pl.ds(p, 16)] = idxv & 7

            def pas(s, c4):
                pa_one(s * 2)
                pa_one(s * 2 + 1)
                for u_ in range(3):
                    j = s * 3 + u_
                    pltpu.async_copy(tab16.at[idxbuf.at[pl.ds(j * 128, 128)]],
                                     rows.at[j], gsem)
                return c4

            lax.fori_loop(0, _NG // 2, pas, 0)

            def pc_one(g):
                sl = g * 16
                acc0 = jnp.zeros((16,), jnp.float32)
                acc1 = jnp.zeros((16,), jnp.float32)
                for r in range(_D1):
                    wv = wbuf[r, pl.ds(sl, 16)]
                    p = sl * 12 + r * 16
                    lov2 = lobuf[pl.ds(p, 16)] * 2
                    i0 = jnp.full((16,), p // 128, jnp.int32)
                    i1 = iota + (p % 128)
                    f0 = plsc.load_gather(rows, [i0, i1, lov2])
                    f1 = plsc.load_gather(rows, [i0, i1, lov2 + 1])
                    acc0 = acc0 + wv * f0
                    acc1 = acc1 + wv * f1
                hc[0, pl.ds(sl, 16)] = acc0
                hc[1, pl.ds(sl, 16)] = acc1

            def pcs(s, c6):
                for u_ in range(3):
                    j = s * 3 + u_
                    pltpu.make_async_copy(tab16.at[idxbuf.at[pl.ds(j * 128, 128)]],
                                          rows.at[j], gsem).wait()
                pc_one(s * 2)
                pc_one(s * 2 + 1)
                return c6

            lax.fori_loop(0, _NG // 2, pcs, 0)
            pltpu.sync_copy(hc, out.at[pl.ds(lev * 2, 2), pl.ds(base_pt, _CP)])
            return c3

        lax.fori_loop(0, _NL, level_body, 0)
        return carry

    lax.fori_loop(0, _CH, chunk_body, 0)


def _sc_encode(x, bidx, z_per_batch, tables):
    xT = x.T
    zT = z_per_batch.T
    tab16 = tables.reshape(_NL * _T // 8, 16)
    mesh = plsc.VectorSubcoreMesh(core_axis_name="c", subcore_axis_name="s")
    kern = pl.kernel(
        _sc_body,
        out_type=jax.ShapeDtypeStruct((2 * _NL, _NPTS), jnp.float32),
        mesh=mesh,
        scratch_types=[
            pltpu.VMEM((3, _CP), jnp.float32),
            pltpu.VMEM((_CP,), jnp.int32),
            pltpu.VMEM((8, 16), jnp.float32),
            pltpu.VMEM((_D1, _CP), jnp.float32),
            pltpu.VMEM((_NTR * 128,), jnp.int32),
            pltpu.VMEM((_NTR * 128,), jnp.int32),
            pltpu.VMEM((_D1, _CP), jnp.float32),
            pltpu.VMEM((_NTR, 128, 16), jnp.float32),
            pltpu.VMEM((256,), jnp.float32),
            pltpu.VMEM((2, _CP), jnp.float32),
            pltpu.SemaphoreType.DMA,
        ],
        compiler_params=pltpu.CompilerParams(needs_layout_passes=False,
                                             use_tc_tiling_on_sc=False),
    )
    return kern(xT, bidx, zT, tab16)


def _mlp_body(h_ref, w0_ref, b0_ref, w1_ref, b1_ref, w2_ref, b2_ref, o_ref):
    hb = h_ref[...]
    a = lax.dot_general(hb, w0_ref[...], (((0,), (0,)), ((), ())),
                        preferred_element_type=jnp.float32) + b0_ref[...]
    a = jnp.maximum(a, 0.0)
    a = jnp.dot(a, w1_ref[...], preferred_element_type=jnp.float32) + b1_ref[...]
    a = jnp.maximum(a, 0.0)
    o_ref[...] = jnp.dot(a, w2_ref[...],
                         preferred_element_type=jnp.float32) + b2_ref[...]


def _mlp_t(hT, W0, b0, W1, b1, W2, b2):
    n = hT.shape[1]
    blk = 2048
    return pl.pallas_call(
        _mlp_body,
        grid=(n // blk,),
        in_specs=[
            pl.BlockSpec((2 * _NL, blk), lambda i: (0, i)),
            pl.BlockSpec((2 * _NL, _WH), lambda i: (0, 0)),
            pl.BlockSpec((1, _WH), lambda i: (0, 0)),
            pl.BlockSpec((_WH, _WH), lambda i: (0, 0)),
            pl.BlockSpec((1, _WH), lambda i: (0, 0)),
            pl.BlockSpec((_WH, _NOUT), lambda i: (0, 0)),
            pl.BlockSpec((1, _NOUT), lambda i: (0, 0)),
        ],
        out_specs=pl.BlockSpec((blk, _NOUT), lambda i: (i, 0)),
        out_shape=jax.ShapeDtypeStruct((n, _NOUT), jnp.float32),
    )(hT, W0, b0.reshape(1, -1), W1, b1.reshape(1, -1), W2, b2.reshape(1, -1))


def kernel(x, bidx, z_per_batch, tables, W0, b0, W1, b1, W2, b2):
    hT = _sc_encode(x, bidx, z_per_batch, tables)
    return _mlp_t(hT, W0, b0, W1, b1, W2, b2)
